# baseline (speedup 1.0000x reference)
import functools

import jax
import jax.numpy as jnp
import numpy as np
from jax import lax
from jax.experimental import pallas as pl
from jax.experimental.pallas import tpu as pltpu
from jax.experimental.pallas import tpu_sc as plsc

_B, _L, _C = 8, 4096, 128
_NP, _PLEN = 32, 256
_NC, _NS = 2, 16
_NW = _NC * _NS

_STARTS = np.array([
    [2244, 1554, 951, 1729, 2189, 1899, 2177, 807, 3334, 1026, 552, 754, 1945, 3291, 2252, 1810, 3403, 2434, 835, 1799, 3382, 2443, 268, 707, 1644, 2321, 752, 1051, 3612, 1079, 1029, 3492],
    [1237, 1838, 2611, 2324, 1582, 2994, 3153, 493, 3079, 3396, 3735, 3709, 1145, 1472, 2876, 164, 3107, 2573, 148, 3035, 3282, 2163, 3064, 1719, 1291, 850, 347, 3001, 25, 1030, 544, 2440],
    [3715, 2937, 820, 1376, 1858, 441, 2476, 2373, 2291, 3373, 3236, 1276, 46, 1450, 305, 2657, 3607, 1744, 437, 556, 177, 824, 600, 1592, 424, 1790, 1119, 661, 2366, 2488, 1939, 3289],
    [3063, 2271, 3770, 1761, 2353, 1372, 1061, 2596, 3199, 1484, 2110, 802, 2457, 2457, 1403, 2815, 291, 188, 577, 2915, 3717, 776, 3166, 2147, 387, 1344, 2, 2883, 1634, 212, 206, 3206],
    [2385, 1372, 535, 3490, 162, 3421, 3823, 3046, 857, 1386, 3281, 1089, 455, 1100, 1435, 2140, 3218, 678, 1579, 2307, 113, 2337, 3271, 1842, 363, 2352, 3232, 1363, 1454, 1937, 1419, 154],
    [814, 852, 2838, 2387, 3214, 1243, 2895, 2335, 3224, 3119, 39, 628, 740, 1761, 1302, 1551, 878, 3528, 3618, 1843, 2564, 3173, 3062, 1543, 1919, 902, 3781, 1656, 172, 2453, 877, 1197],
    [1716, 2445, 343, 211, 1344, 3019, 182, 3006, 1257, 553, 3249, 2405, 3551, 3120, 1218, 98, 1263, 353, 105, 1359, 537, 2996, 1879, 1459, 2045, 3186, 1995, 2809, 1156, 1228, 1777, 1963],
    [1520, 621, 1312, 20, 2396, 52, 2941, 3273, 1183, 3545, 3766, 3243, 488, 3540, 1719, 1381, 3573, 1984, 544, 506, 401, 2937, 21, 216, 576, 1962, 930, 993, 2044, 1767, 1274, 1552],
], dtype=np.int32)

_X_ROWS = _B * _NP * _PLEN
_X_IDX = (
    (np.arange(_B)[:, None, None] * _L + _STARTS[:, :, None]
     + np.arange(_PLEN)[None, None, :])
    .reshape(_X_ROWS // 128, 128)
    .astype(np.int32)
)
_CHUNKS_PER_W = 8
_CHUNK_START = np.zeros((_NW, 16), np.int32)
_CHUNK_START[:, :_CHUNKS_PER_W] = _X_IDX[::2, 0].reshape(_NW, _CHUNKS_PER_W)
_NBUF = 3
_SLAG = 2

_mesh = plsc.VectorSubcoreMesh(
    core_axis_name="c", subcore_axis_name="s", num_cores=_NC, num_subcores=_NS
)


@functools.partial(
    pl.kernel,
    out_type=(
        jax.ShapeDtypeStruct((_X_ROWS * _C,), jnp.float32),
        jax.ShapeDtypeStruct((_X_ROWS // 128, 128), jnp.float32),
    ),
    mesh=_mesh,
    scratch_types=[
        pltpu.VMEM((1, 16), jnp.int32),
        pltpu.VMEM((16, 128), jnp.int32),
        pltpu.VMEM((_NBUF * 256 * _C,), jnp.float32),
        pltpu.VMEM((16, 128), jnp.float32),
        pltpu.SemaphoreType.DMA,
        pltpu.SemaphoreType.DMA,
        pltpu.SemaphoreType.DMA,
    ],
)
def _patch_copy(xf, yf, cstart, yidx, outx, outy, cstart_v, yidx_v, xbuf,
                yrows_v, gsem, ssem, ysem):
    wid = lax.axis_index("s") * _NC + lax.axis_index("c")
    base = wid * 16

    pltpu.sync_copy(cstart.at[pl.ds(wid, 1)], cstart_v)
    pltpu.sync_copy(yidx.at[pl.ds(base, 16)], yidx_v)

    cs = cstart_v[0]
    _CH = 256 * _C

    def _buf(c):
        return xbuf.at[pl.ds((c % _NBUF) * _CH, _CH)]

    def gather(c):
        src = pl.multiple_of(cs[c] * _C, _C)
        pltpu.async_copy(xf.at[pl.ds(src, _CH)], _buf(c), gsem)

    def wait_gather(c):
        src = pl.multiple_of(cs[c] * _C, _C)
        pltpu.make_async_copy(
            xf.at[pl.ds(src, _CH)], _buf(c), gsem
        ).wait()

    def scatter(c):
        dst = pl.multiple_of((wid * _CHUNKS_PER_W + c) * _CH, _CH)
        pltpu.async_copy(_buf(c), outx.at[pl.ds(dst, _CH)], ssem)

    def wait_scatter(c):
        dst = pl.multiple_of((wid * _CHUNKS_PER_W + c) * _CH, _CH)
        pltpu.make_async_copy(
            _buf(c), outx.at[pl.ds(dst, _CH)], ssem
        ).wait()

    for c in range(min(_NBUF - _SLAG + 1, _CHUNKS_PER_W)):
        gather(c)

    for r in range(16):
        pltpu.async_copy(yf.at[yidx_v.at[r]], yrows_v.at[r], ysem)

    last_waited = -1
    for c in range(_CHUNKS_PER_W):
        wait_gather(c)
        scatter(c)
        if c - (_SLAG - 1) >= 0:
            wait_scatter(c - (_SLAG - 1))
            last_waited = c - (_SLAG - 1)
        g = c + _NBUF - _SLAG + 1
        if _NBUF - _SLAG + 1 <= g < _CHUNKS_PER_W:
            gather(g)
    for c in range(last_waited + 1, _CHUNKS_PER_W):
        wait_scatter(c)

    pltpu.make_async_copy(outy.at[pl.ds(0, 16)], yrows_v, ysem).wait()
    pltpu.sync_copy(yrows_v, outy.at[pl.ds(base, 16)])


def kernel(x, y):
    outx, outy = _patch_copy(
        x.reshape(-1),
        y.reshape(-1),
        jnp.asarray(_CHUNK_START),
        jnp.asarray(_X_IDX),
    )
    return (
        outx.reshape(_B, _NP, _PLEN, _C),
        outy.reshape(_B, _NP, _PLEN),
    )


# speedup vs baseline: 1.1532x; 1.1532x over previous
import functools

import jax
import jax.numpy as jnp
import numpy as np
from jax import lax
from jax.experimental import pallas as pl
from jax.experimental.pallas import tpu as pltpu
from jax.experimental.pallas import tpu_sc as plsc

_B, _L, _C = 8, 4096, 128
_NP, _PLEN = 32, 256
_NC, _NS = 2, 16
_NW = _NC * _NS

_STARTS = np.array([
    [2244, 1554, 951, 1729, 2189, 1899, 2177, 807, 3334, 1026, 552, 754, 1945, 3291, 2252, 1810, 3403, 2434, 835, 1799, 3382, 2443, 268, 707, 1644, 2321, 752, 1051, 3612, 1079, 1029, 3492],
    [1237, 1838, 2611, 2324, 1582, 2994, 3153, 493, 3079, 3396, 3735, 3709, 1145, 1472, 2876, 164, 3107, 2573, 148, 3035, 3282, 2163, 3064, 1719, 1291, 850, 347, 3001, 25, 1030, 544, 2440],
    [3715, 2937, 820, 1376, 1858, 441, 2476, 2373, 2291, 3373, 3236, 1276, 46, 1450, 305, 2657, 3607, 1744, 437, 556, 177, 824, 600, 1592, 424, 1790, 1119, 661, 2366, 2488, 1939, 3289],
    [3063, 2271, 3770, 1761, 2353, 1372, 1061, 2596, 3199, 1484, 2110, 802, 2457, 2457, 1403, 2815, 291, 188, 577, 2915, 3717, 776, 3166, 2147, 387, 1344, 2, 2883, 1634, 212, 206, 3206],
    [2385, 1372, 535, 3490, 162, 3421, 3823, 3046, 857, 1386, 3281, 1089, 455, 1100, 1435, 2140, 3218, 678, 1579, 2307, 113, 2337, 3271, 1842, 363, 2352, 3232, 1363, 1454, 1937, 1419, 154],
    [814, 852, 2838, 2387, 3214, 1243, 2895, 2335, 3224, 3119, 39, 628, 740, 1761, 1302, 1551, 878, 3528, 3618, 1843, 2564, 3173, 3062, 1543, 1919, 902, 3781, 1656, 172, 2453, 877, 1197],
    [1716, 2445, 343, 211, 1344, 3019, 182, 3006, 1257, 553, 3249, 2405, 3551, 3120, 1218, 98, 1263, 353, 105, 1359, 537, 2996, 1879, 1459, 2045, 3186, 1995, 2809, 1156, 1228, 1777, 1963],
    [1520, 621, 1312, 20, 2396, 52, 2941, 3273, 1183, 3545, 3766, 3243, 488, 3540, 1719, 1381, 3573, 1984, 544, 506, 401, 2937, 21, 216, 576, 1962, 930, 993, 2044, 1767, 1274, 1552],
], dtype=np.int32)

_X_ROWS = _B * _NP * _PLEN
_X_IDX = (
    (np.arange(_B)[:, None, None] * _L + _STARTS[:, :, None]
     + np.arange(_PLEN)[None, None, :])
    .reshape(_X_ROWS // 128, 128)
    .astype(np.int32)
)
_CHUNKS_PER_W = (_X_ROWS // 128) // _NW
_PATCH_PER_W = (_B * _NP) // _NW
_PATCH_START = np.zeros((_NW, 16), np.int32)
_PATCH_START[:, :_PATCH_PER_W] = (
    np.arange(_B)[:, None] * _L + _STARTS
).reshape(_NW, _PATCH_PER_W).astype(np.int32)
_YWIN = 272
_NBUF = 7
_SLAG = 3

_mesh = plsc.VectorSubcoreMesh(
    core_axis_name="c", subcore_axis_name="s", num_cores=_NC, num_subcores=_NS
)


@functools.partial(
    pl.kernel,
    out_type=(
        jax.ShapeDtypeStruct((_X_ROWS, _C), jnp.float32),
        jax.ShapeDtypeStruct((_X_ROWS,), jnp.float32),
    ),
    mesh=_mesh,
    scratch_types=[
        pltpu.VMEM((_CHUNKS_PER_W, 128), jnp.int32),
        pltpu.VMEM((1, 16), jnp.int32),
        pltpu.VMEM((_NBUF, 128, _C), jnp.float32),
        pltpu.VMEM((_PATCH_PER_W * _YWIN,), jnp.float32),
        pltpu.VMEM((_PATCH_PER_W * _PLEN,), jnp.float32),
        pltpu.SemaphoreType.DMA,
        pltpu.SemaphoreType.DMA,
        pltpu.SemaphoreType.DMA,
    ],
)
def _patch_copy(x2d, yf, xidx, pstart, outx, outy, xidx_v, pstart_v, xbuf,
                ywin_v, yout_v, gsem, ssem, ysem):
    wid = lax.axis_index("s") * _NC + lax.axis_index("c")
    base = wid * _CHUNKS_PER_W

    pltpu.sync_copy(xidx.at[pl.ds(base, _CHUNKS_PER_W)], xidx_v)
    pltpu.sync_copy(pstart.at[pl.ds(wid, 1)], pstart_v)
    ps = pstart_v[0]
    ps_al = ps & -8
    ps_skew = ps - ps_al

    def gather(c):
        pltpu.async_copy(x2d.at[xidx_v.at[c]], xbuf.at[c % _NBUF], gsem)

    def wait_gather(c):
        pltpu.make_async_copy(
            x2d.at[xidx_v.at[c]], xbuf.at[c % _NBUF], gsem
        ).wait()

    def scatter(c):
        pltpu.async_copy(
            xbuf.at[c % _NBUF],
            outx.at[pl.ds(base * 128 + c * 128, 128)],
            ssem,
        )

    def wait_scatter(c):
        pltpu.make_async_copy(
            xbuf.at[c % _NBUF],
            outx.at[pl.ds(base * 128 + c * 128, 128)],
            ssem,
        ).wait()

    for c in range(min(_NBUF - _SLAG + 1, _CHUNKS_PER_W)):
        gather(c)

    for j in range(_PATCH_PER_W):
        o_al = pl.multiple_of(ps_al[j], 8)
        pltpu.async_copy(
            yf.at[pl.ds(o_al, _YWIN)],
            ywin_v.at[pl.ds(j * _YWIN, _YWIN)],
            ysem,
        )

    last_waited = -1
    for c in range(_CHUNKS_PER_W):
        wait_gather(c)
        scatter(c)
        if c - (_SLAG - 1) >= 0:
            wait_scatter(c - (_SLAG - 1))
            last_waited = c - (_SLAG - 1)
        g = c + _NBUF - _SLAG + 1
        if _NBUF - _SLAG + 1 <= g < _CHUNKS_PER_W:
            gather(g)
    for c in range(last_waited + 1, _CHUNKS_PER_W):
        wait_scatter(c)

    pltpu.make_async_copy(
        yf.at[pl.ds(0, _PATCH_PER_W * _YWIN)], ywin_v, ysem
    ).wait()
    for j in range(_PATCH_PER_W):
        r = ps_skew[j]
        for t in range(_PLEN // 16):
            seg = ywin_v[pl.ds(j * _YWIN + t * 16 + r, 16)]
            yout_v[pl.ds(j * _PLEN + t * 16, 16)] = seg
    pltpu.sync_copy(yout_v, outy.at[pl.ds(wid * _PATCH_PER_W * _PLEN, _PATCH_PER_W * _PLEN)])


def kernel(x, y):
    outx, outy = _patch_copy(
        x.reshape(_B * _L, _C),
        y.reshape(-1),
        jnp.asarray(_X_IDX),
        jnp.asarray(_PATCH_START),
    )
    return (
        outx.reshape(_B, _NP, _PLEN, _C),
        outy.reshape(_B, _NP, _PLEN),
    )


# speedup vs baseline: 1.1602x; 1.0060x over previous
import functools

import jax
import jax.numpy as jnp
import numpy as np
from jax import lax
from jax.experimental import pallas as pl
from jax.experimental.pallas import tpu as pltpu
from jax.experimental.pallas import tpu_sc as plsc

_B, _L, _C = 8, 4096, 128
_NP, _PLEN = 32, 256
_NC, _NS = 2, 16
_NW = _NC * _NS

_STARTS = np.array([
    [2244, 1554, 951, 1729, 2189, 1899, 2177, 807, 3334, 1026, 552, 754, 1945, 3291, 2252, 1810, 3403, 2434, 835, 1799, 3382, 2443, 268, 707, 1644, 2321, 752, 1051, 3612, 1079, 1029, 3492],
    [1237, 1838, 2611, 2324, 1582, 2994, 3153, 493, 3079, 3396, 3735, 3709, 1145, 1472, 2876, 164, 3107, 2573, 148, 3035, 3282, 2163, 3064, 1719, 1291, 850, 347, 3001, 25, 1030, 544, 2440],
    [3715, 2937, 820, 1376, 1858, 441, 2476, 2373, 2291, 3373, 3236, 1276, 46, 1450, 305, 2657, 3607, 1744, 437, 556, 177, 824, 600, 1592, 424, 1790, 1119, 661, 2366, 2488, 1939, 3289],
    [3063, 2271, 3770, 1761, 2353, 1372, 1061, 2596, 3199, 1484, 2110, 802, 2457, 2457, 1403, 2815, 291, 188, 577, 2915, 3717, 776, 3166, 2147, 387, 1344, 2, 2883, 1634, 212, 206, 3206],
    [2385, 1372, 535, 3490, 162, 3421, 3823, 3046, 857, 1386, 3281, 1089, 455, 1100, 1435, 2140, 3218, 678, 1579, 2307, 113, 2337, 3271, 1842, 363, 2352, 3232, 1363, 1454, 1937, 1419, 154],
    [814, 852, 2838, 2387, 3214, 1243, 2895, 2335, 3224, 3119, 39, 628, 740, 1761, 1302, 1551, 878, 3528, 3618, 1843, 2564, 3173, 3062, 1543, 1919, 902, 3781, 1656, 172, 2453, 877, 1197],
    [1716, 2445, 343, 211, 1344, 3019, 182, 3006, 1257, 553, 3249, 2405, 3551, 3120, 1218, 98, 1263, 353, 105, 1359, 537, 2996, 1879, 1459, 2045, 3186, 1995, 2809, 1156, 1228, 1777, 1963],
    [1520, 621, 1312, 20, 2396, 52, 2941, 3273, 1183, 3545, 3766, 3243, 488, 3540, 1719, 1381, 3573, 1984, 544, 506, 401, 2937, 21, 216, 576, 1962, 930, 993, 2044, 1767, 1274, 1552],
], dtype=np.int32)

_X_ROWS = _B * _NP * _PLEN
_X_IDX = (
    (np.arange(_B)[:, None, None] * _L + _STARTS[:, :, None]
     + np.arange(_PLEN)[None, None, :])
    .reshape(_X_ROWS // 128, 128)
    .astype(np.int32)
)
_CHUNKS_PER_W = (_X_ROWS // 128) // _NW
_PATCH_PER_W = (_B * _NP) // _NW
_PATCH_START = np.zeros((_NW, 16), np.int32)
_PATCH_START[:, :_PATCH_PER_W] = (
    np.arange(_B)[:, None] * _L + _STARTS
).reshape(_NW, _PATCH_PER_W).astype(np.int32)
_YWIN = 272
_NBUF = 7
_SLAG = 5

_mesh = plsc.VectorSubcoreMesh(
    core_axis_name="c", subcore_axis_name="s", num_cores=_NC, num_subcores=_NS
)


@functools.partial(
    pl.kernel,
    out_type=(
        jax.ShapeDtypeStruct((_X_ROWS, _C), jnp.float32),
        jax.ShapeDtypeStruct((_X_ROWS,), jnp.float32),
    ),
    mesh=_mesh,
    scratch_types=[
        pltpu.VMEM((_CHUNKS_PER_W, 128), jnp.int32),
        pltpu.VMEM((1, 16), jnp.int32),
        pltpu.VMEM((_NBUF, 128, _C), jnp.float32),
        pltpu.VMEM((_PATCH_PER_W * _YWIN,), jnp.float32),
        pltpu.VMEM((_PATCH_PER_W * _PLEN,), jnp.float32),
        pltpu.SemaphoreType.DMA,
        pltpu.SemaphoreType.DMA,
        pltpu.SemaphoreType.DMA,
    ],
)
def _patch_copy(x2d, yf, xidx, pstart, outx, outy, xidx_v, pstart_v, xbuf,
                ywin_v, yout_v, gsem, ssem, ysem):
    wid = lax.axis_index("s") * _NC + lax.axis_index("c")
    base = wid * _CHUNKS_PER_W

    pltpu.sync_copy(xidx.at[pl.ds(base, _CHUNKS_PER_W)], xidx_v)
    pltpu.sync_copy(pstart.at[pl.ds(wid, 1)], pstart_v)
    ps = pstart_v[0]
    ps_al = ps & -8
    ps_skew = ps - ps_al

    def gather(c):
        pltpu.async_copy(x2d.at[xidx_v.at[c]], xbuf.at[c % _NBUF], gsem)

    def wait_gather(c):
        pltpu.make_async_copy(
            x2d.at[xidx_v.at[c]], xbuf.at[c % _NBUF], gsem
        ).wait()

    def scatter(c):
        pltpu.async_copy(
            xbuf.at[c % _NBUF],
            outx.at[pl.ds(base * 128 + c * 128, 128)],
            ssem,
        )

    def wait_scatter(c):
        pltpu.make_async_copy(
            xbuf.at[c % _NBUF],
            outx.at[pl.ds(base * 128 + c * 128, 128)],
            ssem,
        ).wait()

    for c in range(min(_NBUF - _SLAG + 1, _CHUNKS_PER_W)):
        gather(c)

    for j in range(_PATCH_PER_W):
        o_al = pl.multiple_of(ps_al[j], 8)
        pltpu.async_copy(
            yf.at[pl.ds(o_al, _YWIN)],
            ywin_v.at[pl.ds(j * _YWIN, _YWIN)],
            ysem,
        )

    last_waited = -1
    for c in range(_CHUNKS_PER_W):
        wait_gather(c)
        scatter(c)
        if c - (_SLAG - 1) >= 0:
            wait_scatter(c - (_SLAG - 1))
            last_waited = c - (_SLAG - 1)
        g = c + _NBUF - _SLAG + 1
        if _NBUF - _SLAG + 1 <= g < _CHUNKS_PER_W:
            gather(g)
    for c in range(last_waited + 1, _CHUNKS_PER_W):
        wait_scatter(c)

    pltpu.make_async_copy(
        yf.at[pl.ds(0, _PATCH_PER_W * _YWIN)], ywin_v, ysem
    ).wait()
    for j in range(_PATCH_PER_W):
        r = ps_skew[j]
        for t in range(_PLEN // 16):
            seg = ywin_v[pl.ds(j * _YWIN + t * 16 + r, 16)]
            yout_v[pl.ds(j * _PLEN + t * 16, 16)] = seg
    pltpu.sync_copy(yout_v, outy.at[pl.ds(wid * _PATCH_PER_W * _PLEN, _PATCH_PER_W * _PLEN)])


def kernel(x, y):
    outx, outy = _patch_copy(
        x.reshape(_B * _L, _C),
        y.reshape(-1),
        jnp.asarray(_X_IDX),
        jnp.asarray(_PATCH_START),
    )
    return (
        outx.reshape(_B, _NP, _PLEN, _C),
        outy.reshape(_B, _NP, _PLEN),
    )


# speedup vs baseline: 1.1696x; 1.0081x over previous
import functools

import jax
import jax.numpy as jnp
import numpy as np
from jax import lax
from jax.experimental import pallas as pl
from jax.experimental.pallas import tpu as pltpu
from jax.experimental.pallas import tpu_sc as plsc

_B, _L, _C = 8, 4096, 128
_NP, _PLEN = 32, 256
_NC, _NS = 2, 16
_NW = _NC * _NS

_STARTS = np.array([
    [2244, 1554, 951, 1729, 2189, 1899, 2177, 807, 3334, 1026, 552, 754, 1945, 3291, 2252, 1810, 3403, 2434, 835, 1799, 3382, 2443, 268, 707, 1644, 2321, 752, 1051, 3612, 1079, 1029, 3492],
    [1237, 1838, 2611, 2324, 1582, 2994, 3153, 493, 3079, 3396, 3735, 3709, 1145, 1472, 2876, 164, 3107, 2573, 148, 3035, 3282, 2163, 3064, 1719, 1291, 850, 347, 3001, 25, 1030, 544, 2440],
    [3715, 2937, 820, 1376, 1858, 441, 2476, 2373, 2291, 3373, 3236, 1276, 46, 1450, 305, 2657, 3607, 1744, 437, 556, 177, 824, 600, 1592, 424, 1790, 1119, 661, 2366, 2488, 1939, 3289],
    [3063, 2271, 3770, 1761, 2353, 1372, 1061, 2596, 3199, 1484, 2110, 802, 2457, 2457, 1403, 2815, 291, 188, 577, 2915, 3717, 776, 3166, 2147, 387, 1344, 2, 2883, 1634, 212, 206, 3206],
    [2385, 1372, 535, 3490, 162, 3421, 3823, 3046, 857, 1386, 3281, 1089, 455, 1100, 1435, 2140, 3218, 678, 1579, 2307, 113, 2337, 3271, 1842, 363, 2352, 3232, 1363, 1454, 1937, 1419, 154],
    [814, 852, 2838, 2387, 3214, 1243, 2895, 2335, 3224, 3119, 39, 628, 740, 1761, 1302, 1551, 878, 3528, 3618, 1843, 2564, 3173, 3062, 1543, 1919, 902, 3781, 1656, 172, 2453, 877, 1197],
    [1716, 2445, 343, 211, 1344, 3019, 182, 3006, 1257, 553, 3249, 2405, 3551, 3120, 1218, 98, 1263, 353, 105, 1359, 537, 2996, 1879, 1459, 2045, 3186, 1995, 2809, 1156, 1228, 1777, 1963],
    [1520, 621, 1312, 20, 2396, 52, 2941, 3273, 1183, 3545, 3766, 3243, 488, 3540, 1719, 1381, 3573, 1984, 544, 506, 401, 2937, 21, 216, 576, 1962, 930, 993, 2044, 1767, 1274, 1552],
], dtype=np.int32)

_X_ROWS = _B * _NP * _PLEN
_X_IDX = (
    (np.arange(_B)[:, None, None] * _L + _STARTS[:, :, None]
     + np.arange(_PLEN)[None, None, :])
    .reshape(_X_ROWS // 128, 128)
    .astype(np.int32)
)
_CHUNKS_PER_W = (_X_ROWS // 128) // _NW
_PATCH_PER_W = (_B * _NP) // _NW
_PATCH_START = np.zeros((_NW, 16), np.int32)
_PATCH_START[:, :_PATCH_PER_W] = (
    np.arange(_B)[:, None] * _L + _STARTS
).reshape(_NW, _PATCH_PER_W).astype(np.int32)
_YWIN = 272
_NBUF = 7
_SLAG = 5

_mesh = plsc.VectorSubcoreMesh(
    core_axis_name="c", subcore_axis_name="s", num_cores=_NC, num_subcores=_NS
)


@functools.partial(
    pl.kernel,
    out_type=(
        jax.ShapeDtypeStruct((_X_ROWS, _C), jnp.float32),
        jax.ShapeDtypeStruct((_X_ROWS,), jnp.float32),
    ),
    mesh=_mesh,
    scratch_types=[
        pltpu.VMEM((_CHUNKS_PER_W, 128), jnp.int32),
        pltpu.VMEM((1, 16), jnp.int32),
        pltpu.VMEM((_NBUF, 128, _C), jnp.float32),
        pltpu.VMEM((_PATCH_PER_W * _YWIN,), jnp.float32),
        pltpu.VMEM((_PATCH_PER_W * _PLEN,), jnp.float32),
        pltpu.SemaphoreType.DMA,
        pltpu.SemaphoreType.DMA,
        pltpu.SemaphoreType.DMA,
    ],
)
def _patch_copy(x2d, yf, xidx, pstart, outx, outy, xidx_v, pstart_v, xbuf,
                ywin_v, yout_v, gsem, ssem, ysem):
    wid = lax.axis_index("s") * _NC + lax.axis_index("c")
    base = wid * _CHUNKS_PER_W

    pltpu.sync_copy(xidx.at[pl.ds(base, _CHUNKS_PER_W)], xidx_v)
    pltpu.sync_copy(pstart.at[pl.ds(wid, 1)], pstart_v)
    ps = pstart_v[0]
    ps_al = ps & -8
    ps_skew = ps - ps_al

    def gather(c):
        pltpu.async_copy(x2d.at[xidx_v.at[c]], xbuf.at[c % _NBUF], gsem)

    def wait_gather(c):
        pltpu.make_async_copy(
            x2d.at[xidx_v.at[c]], xbuf.at[c % _NBUF], gsem
        ).wait()

    def scatter(c):
        pltpu.async_copy(
            xbuf.at[c % _NBUF],
            outx.at[pl.ds(base * 128 + c * 128, 128)],
            ssem,
        )

    def wait_scatter(c):
        pltpu.make_async_copy(
            xbuf.at[c % _NBUF],
            outx.at[pl.ds(base * 128 + c * 128, 128)],
            ssem,
        ).wait()

    for c in range(min(_NBUF - _SLAG + 1, _CHUNKS_PER_W)):
        gather(c)

    for j in range(_PATCH_PER_W):
        o_al = pl.multiple_of(ps_al[j], 8)
        pltpu.async_copy(
            yf.at[pl.ds(o_al, _YWIN)],
            ywin_v.at[pl.ds(j * _YWIN, _YWIN)],
            ysem,
        )

    last_waited = -1
    for c in range(_CHUNKS_PER_W):
        wait_gather(c)
        scatter(c)
        if c - (_SLAG - 1) >= 0:
            wait_scatter(c - (_SLAG - 1))
            last_waited = c - (_SLAG - 1)
        g = c + _NBUF - _SLAG + 1
        if _NBUF - _SLAG + 1 <= g < _CHUNKS_PER_W:
            gather(g)
    pltpu.make_async_copy(
        yf.at[pl.ds(0, _PATCH_PER_W * _YWIN)], ywin_v, ysem
    ).wait()
    for j in range(_PATCH_PER_W):
        r = ps_skew[j]
        for t in range(_PLEN // 16):
            seg = ywin_v[pl.ds(j * _YWIN + t * 16 + r, 16)]
            yout_v[pl.ds(j * _PLEN + t * 16, 16)] = seg
    yo = outy.at[pl.ds(wid * _PATCH_PER_W * _PLEN, _PATCH_PER_W * _PLEN)]
    pltpu.async_copy(yout_v, yo, ysem)
    for c in range(last_waited + 1, _CHUNKS_PER_W):
        wait_scatter(c)
    pltpu.make_async_copy(yout_v, yo, ysem).wait()


def kernel(x, y):
    outx, outy = _patch_copy(
        x.reshape(_B * _L, _C),
        y.reshape(-1),
        jnp.asarray(_X_IDX),
        jnp.asarray(_PATCH_START),
    )
    return (
        outx.reshape(_B, _NP, _PLEN, _C),
        outy.reshape(_B, _NP, _PLEN),
    )
